# Initial kernel scaffold; baseline (speedup 1.0000x reference)
import jax
import jax.numpy as jnp
from jax.experimental import pallas as pl


def kernel(x, edge_index, edge_weight, batch, W1, b1, W2, b2):
    raise NotImplementedError("write your pallas kernel here")


# speedup vs baseline: 10.7614x; 10.7614x over previous
import functools

import jax
import jax.numpy as jnp
from jax import lax
from jax.experimental import pallas as pl
from jax.experimental.pallas import tpu as pltpu
from jax.experimental.pallas import tpu_sc as plsc

N = 10000
E = 320000
D = 128
G = 128

NC = 2
NS = 16
NW = NC * NS
L = 16

CK = 128
EPT = -(-E // NW)
CHUNKS = -(-EPT // CK)
E_PAD = NW * CHUNKS * CK

NP = 10240
RPT = NP // NS
EC = NP // CK
CPT = -(-EC // NW)
ES = (G + 1) * D

_mesh = plsc.VectorSubcoreMesh(core_axis_name="c", subcore_axis_name="s")


_GD = lax.GatherDimensionNumbers(
    offset_dims=(), collapsed_slice_dims=(0,), start_index_map=(0,))


def _bcast_lane(vec, i):
    idx = jnp.full((L, 1), i, jnp.int32)
    return lax.gather(vec, idx, _GD, (1,),
                      mode=lax.GatherScatterMode.PROMISE_IN_BOUNDS)


@functools.partial(
    pl.kernel,
    out_type=jax.ShapeDtypeStruct((NC, NP), jnp.float32),
    mesh=_mesh,
    compiler_params=pltpu.CompilerParams(needs_layout_passes=False),
    scratch_types=[
        pltpu.VMEM((CHUNKS, CK), jnp.int32),
        pltpu.VMEM((CHUNKS, CK), jnp.float32),
        pltpu.VMEM((RPT,), jnp.float32),
        pltpu.VMEM_SHARED((NP,), jnp.float32),
    ],
)
def _deg_kernel(dst_h, w_h, deg_out, dst_v, w_v, zbuf, deg_sh):
    c = lax.axis_index("c")
    s = lax.axis_index("s")
    wid = c * NS + s
    zv = jnp.zeros((L,), jnp.float32)
    for i in range(RPT // L):
        zbuf[pl.ds(i * L, L)] = zv
    pltpu.sync_copy(zbuf, deg_sh.at[pl.ds(s * RPT, RPT)])
    plsc.subcore_barrier()
    pltpu.sync_copy(dst_h.at[wid], dst_v)
    pltpu.sync_copy(w_h.at[wid], w_v)

    def chunk(j, carry):
        pltpu.sync_copy(w_v.at[j], deg_sh.at[dst_v.at[j]], add=True)
        return carry

    lax.fori_loop(0, CHUNKS, chunk, 0)
    plsc.subcore_barrier()
    pltpu.sync_copy(deg_sh.at[pl.ds(s * RPT, RPT)],
                    deg_out.at[c, pl.ds(s * RPT, RPT)])


@functools.partial(
    pl.kernel,
    out_type=jax.ShapeDtypeStruct((NC, NP, D), jnp.float32),
    mesh=_mesh,
    compiler_params=pltpu.CompilerParams(needs_layout_passes=False),
    scratch_types=[
        pltpu.VMEM((CHUNKS, CK), jnp.int32),
        pltpu.VMEM((CHUNKS, CK), jnp.int32),
        pltpu.VMEM((CHUNKS, CK), jnp.float32),
        pltpu.VMEM((CK, D), jnp.float32),
        pltpu.VMEM_SHARED((NP, D), jnp.float32),
        pltpu.SemaphoreType.DMA,
    ],
)
def _scatter_kernel(xs_h, src_h, dst_h, w_h, acc_out,
                    src_v, dst_v, w_v, rows_v, acc_sh, sem):
    c = lax.axis_index("c")
    s = lax.axis_index("s")
    wid = c * NS + s
    zv = jnp.zeros((L,), jnp.float32)

    def zrow(r, carry):
        for f in range(D // L):
            rows_v[r, pl.ds(f * L, L)] = zv
        return carry

    lax.fori_loop(0, CK, zrow, 0)
    for k in range(RPT // CK):
        pltpu.sync_copy(rows_v, acc_sh.at[pl.ds(s * RPT + k * CK, CK)])
    plsc.subcore_barrier()
    pltpu.sync_copy(src_h.at[wid], src_v)
    pltpu.sync_copy(dst_h.at[wid], dst_v)
    pltpu.sync_copy(w_h.at[wid], w_v)

    def chunk(j, carry):
        pltpu.async_copy(xs_h.at[src_v.at[j]], rows_v, sem).wait()

        def group(m, gcarry):
            w16 = w_v[j, pl.ds(m * L, L)]
            for i in range(L):
                wb = _bcast_lane(w16, i)
                e = m * L + i
                for f in range(D // L):
                    sl = pl.ds(f * L, L)
                    rows_v[e, sl] = rows_v[e, sl] * wb
            return gcarry

        lax.fori_loop(0, CK // L, group, 0)
        pltpu.sync_copy(rows_v, acc_sh.at[dst_v.at[j]], add=True)
        return carry

    lax.fori_loop(0, CHUNKS, chunk, 0)
    plsc.subcore_barrier()
    for k in range(RPT // CK):
        sl = pl.ds(s * RPT + k * CK, CK)
        pltpu.sync_copy(acc_sh.at[sl], acc_out.at[c, sl])


@functools.partial(
    pl.kernel,
    out_type=jax.ShapeDtypeStruct((NW, G + 1, D), jnp.float32),
    mesh=_mesh,
    compiler_params=pltpu.CompilerParams(needs_layout_passes=False),
    scratch_types=[
        pltpu.VMEM((G + 1, D), jnp.float32),
        pltpu.VMEM((CK, D), jnp.float32),
        pltpu.VMEM((CK,), jnp.int32),
    ],
)
def _segmax_kernel(h_h, batch_h, emb_out, emb_v, buf, bid_v):
    c = lax.axis_index("c")
    s = lax.axis_index("s")
    wid = c * NS + s
    ninf = jnp.full((L,), -jnp.inf, jnp.float32)

    def init(i, carry):
        for f in range(D // L):
            emb_v[i, pl.ds(f * L, L)] = ninf
        return carry

    lax.fori_loop(0, G + 1, init, 0)
    offs = [lax.iota(jnp.int32, L) + f * L for f in range(D // L)]
    for k in range(CPT):
        cid = wid * CPT + k

        @pl.when(cid < EC)
        def _():
            pltpu.sync_copy(h_h.at[pl.ds(cid * CK, CK)], buf)
            pltpu.sync_copy(batch_h.at[pl.ds(cid * CK, CK)], bid_v)

            def group(m, gcarry):
                bids = bid_v[pl.ds(m * L, L)]
                for i in range(L):
                    b16 = _bcast_lane(bids, i)
                    r = m * L + i
                    for f in range(D // L):
                        cur = plsc.load_gather(emb_v, [b16, offs[f]])
                        val = jnp.maximum(cur, buf[r, pl.ds(f * L, L)])
                        plsc.store_scatter(emb_v, [b16, offs[f]], val)
                return gcarry

            lax.fori_loop(0, CK // L, group, 0)

    pltpu.sync_copy(emb_v, emb_out.at[wid])


def _pre_body(deg_ref, x_ref, w_ref, xw_ref, xs_ref):
    deg = deg_ref[:, 0:1] + deg_ref[:, 1:2] + 1.0
    dis = lax.rsqrt(deg)
    xw = jnp.dot(x_ref[...], w_ref[...], preferred_element_type=jnp.float32)
    xw_ref[...] = xw
    xs_ref[...] = xw * dis


def _gelu(pre):
    return 0.5 * pre * (1.0 + lax.erf(pre * 0.7071067811865476))


def _mid_body(acc_ref, xw_ref, deg_ref, b_ref, w2_ref, out_ref, xw2_ref, xs2_ref):
    dis = lax.rsqrt(deg_ref[:, 0:1] + deg_ref[:, 1:2] + 1.0)
    a = acc_ref[0] + acc_ref[1]
    pre = dis * (a + dis * xw_ref[...]) + b_ref[...]
    o = _gelu(pre)
    out_ref[...] = o
    xw2 = jnp.dot(o, w2_ref[...], preferred_element_type=jnp.float32)
    xw2_ref[...] = xw2
    xs2_ref[...] = xw2 * dis


def _post_body(acc_ref, xw_ref, deg_ref, b_ref, out_ref):
    dis = lax.rsqrt(deg_ref[:, 0:1] + deg_ref[:, 1:2] + 1.0)
    a = acc_ref[0] + acc_ref[1]
    pre = dis * (a + dis * xw_ref[...]) + b_ref[...]
    out_ref[...] = _gelu(pre)


def _combine_body(e1_ref, e2_ref, o1_ref, o2_ref):
    acc1 = e1_ref[pl.ds(0, G), :]
    acc2 = e2_ref[pl.ds(0, G), :]
    for i in range(1, NW):
        acc1 = jnp.maximum(acc1, e1_ref[pl.ds(i * (G + 1), G), :])
        acc2 = jnp.maximum(acc2, e2_ref[pl.ds(i * (G + 1), G), :])
    o1_ref[...] = acc1
    o2_ref[...] = acc2


def kernel(x, edge_index, edge_weight, batch, W1, b1, W2, b2):
    f32 = jnp.float32
    src = edge_index[0]
    dst = edge_index[1]
    pad_e = E_PAD - E
    srcp = jnp.pad(src, (0, pad_e)).reshape(NW, CHUNKS, CK)
    dstp = jnp.pad(dst, (0, pad_e)).reshape(NW, CHUNKS, CK)
    wp = jnp.pad(edge_weight, (0, pad_e)).reshape(NW, CHUNKS, CK)
    x_pad = jnp.pad(x, ((0, NP - N), (0, 0)))
    batch_pad = jnp.pad(batch, (0, NP - N), constant_values=G)
    b1r = b1.reshape(1, D)
    b2r = b2.reshape(1, D)

    deg_parts = _deg_kernel(dstp, wp)
    deg_t = deg_parts.T

    xw1, xs1 = pl.pallas_call(
        _pre_body,
        out_shape=(jax.ShapeDtypeStruct((NP, D), f32),
                   jax.ShapeDtypeStruct((NP, D), f32)),
    )(deg_t, x_pad, W1)

    acc1 = _scatter_kernel(xs1, srcp, dstp, wp)

    out1, xw2, xs2 = pl.pallas_call(
        _mid_body,
        out_shape=(jax.ShapeDtypeStruct((NP, D), f32),
                   jax.ShapeDtypeStruct((NP, D), f32),
                   jax.ShapeDtypeStruct((NP, D), f32)),
    )(acc1, xw1, deg_t, b1r, W2)

    emb1p = _segmax_kernel(out1, batch_pad)

    acc2 = _scatter_kernel(xs2, srcp, dstp, wp)

    out2 = pl.pallas_call(
        _post_body,
        out_shape=jax.ShapeDtypeStruct((NP, D), f32),
    )(acc2, xw2, deg_t, b2r)

    emb2p = _segmax_kernel(out2, batch_pad)

    emb1, emb2 = pl.pallas_call(
        _combine_body,
        out_shape=(jax.ShapeDtypeStruct((G, D), f32),
                   jax.ShapeDtypeStruct((G, D), f32)),
    )(emb1p.reshape(NW * (G + 1), D), emb2p.reshape(NW * (G + 1), D))
    return (emb1, emb2)
